# Initial kernel scaffold; baseline (speedup 1.0000x reference)
import jax
import jax.numpy as jnp
from jax.experimental import pallas as pl


def kernel(rules, relation, table):
    raise NotImplementedError("write your pallas kernel here")


# speedup vs baseline: 181.6839x; 181.6839x over previous
import functools

import jax
import jax.numpy as jnp
from jax import lax
from jax.experimental import pallas as pl
from jax.experimental.pallas import tpu as pltpu
from jax.experimental.pallas import tpu_sc as plsc

B = 16384
H = 50
LEN_RULES = 100000
PAD_TOK = LEN_RULES
TBL_PAD = 100008
NC, NS, L = 2, 16, 16
NW = NC * NS
ROWS_PER_W = B // NW
IDX_PER_W = ROWS_PER_W * H
GROUPS = ROWS_PER_W // L
NACC = 4


def _body(rules_hbm, table_hbm, out_hbm, table_v, idx_v, out_v):
    wid = lax.axis_index("s") * NC + lax.axis_index("c")
    pltpu.sync_copy(table_hbm, table_v)
    pltpu.sync_copy(rules_hbm.at[pl.ds(wid * IDX_PER_W, IDX_PER_W)], idx_v)

    lanes = lax.iota(jnp.int32, L)

    def group(g, _):
        pos0 = (g * L + lanes) * H
        acc = [jnp.ones((L,), jnp.float32) for _ in range(NACC)]
        for j in range(H):
            iv = plsc.load_gather(idx_v, [pos0 + j])
            v = plsc.load_gather(table_v, [iv])
            acc[j % NACC] = acc[j % NACC] * (1.0 + jnp.exp(v))
        p = (acc[0] * acc[1]) * (acc[2] * acc[3])
        no = 1.0 - 1.0 / p
        no = jnp.minimum(jnp.maximum(no, 0.0001), 0.99999)
        out_v[pl.ds(g * L, L)] = no
        return 0

    lax.fori_loop(0, GROUPS, group, 0)
    pltpu.sync_copy(out_v, out_hbm.at[pl.ds(wid * ROWS_PER_W, ROWS_PER_W)])


@jax.jit
def kernel(rules, relation, table):
    del relation
    tbl = table[:, 0].at[PAD_TOK].set(-jnp.inf)
    tbl = jnp.concatenate([tbl, jnp.zeros((TBL_PAD - (LEN_RULES + 1),), jnp.float32)])
    rules_flat = rules.reshape(-1).astype(jnp.int32)

    run = pl.kernel(
        _body,
        out_type=jax.ShapeDtypeStruct((B,), jnp.float32),
        mesh=plsc.VectorSubcoreMesh(
            core_axis_name="c", subcore_axis_name="s",
            num_cores=NC, num_subcores=NS,
        ),
        compiler_params=pltpu.CompilerParams(needs_layout_passes=False),
        scratch_types=[
            pltpu.VMEM((TBL_PAD,), jnp.float32),
            pltpu.VMEM((IDX_PER_W,), jnp.int32),
            pltpu.VMEM((ROWS_PER_W,), jnp.float32),
        ],
    )
    return run(rules_flat, tbl).reshape(B, 1)
